# baseline (speedup 1.0000x reference)
import functools

import jax
import jax.numpy as jnp
from jax.experimental import pallas as pl
from jax.experimental.pallas import tpu as pltpu

_D_MODEL = 2048
_NUM_EXPERTS = 8
_TOP_K = 2
_AUX_LOSS_WEIGHT = 0.01
_DPSL_PRIOR = 0.125

_BT = 1024


def _router_body(num_blocks, total_tokens,
                 x_ref, wt_ref,
                 probs_ref, idx_ref, topp_ref, aux_ref,
                 acc_p, acc_c):
    i = pl.program_id(0)

    @pl.when(i == 0)
    def _init():
        acc_p[...] = jnp.zeros_like(acc_p)
        acc_c[...] = jnp.zeros_like(acc_c)

    logits = jnp.dot(x_ref[...], wt_ref[...],
                     preferred_element_type=jnp.float32)
    m = jnp.max(logits, axis=-1, keepdims=True)
    e = jnp.exp(logits - m)
    s = jnp.sum(e, axis=-1, keepdims=True)
    probs = e / s
    probs_ref[...] = probs

    iota = jax.lax.broadcasted_iota(jnp.int32, probs.shape, 1)
    p1 = jnp.max(probs, axis=-1, keepdims=True)
    i1 = jnp.min(jnp.where(probs == p1, iota, _NUM_EXPERTS),
                 axis=-1, keepdims=True)
    masked = jnp.where(iota == i1, -jnp.inf, probs)
    p2 = jnp.max(masked, axis=-1, keepdims=True)
    i2 = jnp.min(jnp.where(masked == p2, iota, _NUM_EXPERTS),
                 axis=-1, keepdims=True)
    denom = p1 + p2
    idx_ref[...] = jnp.concatenate([i1, i2], axis=-1)
    topp_ref[...] = jnp.concatenate([p1 / denom, p2 / denom], axis=-1)

    acc_p[...] += jnp.sum(probs, axis=0, keepdims=True)
    cnt = ((iota == i1).astype(jnp.float32)
           + (iota == i2).astype(jnp.float32))
    acc_c[...] += jnp.sum(cnt, axis=0, keepdims=True)

    @pl.when(i == num_blocks - 1)
    def _finish():
        inv_t = 1.0 / total_tokens
        P_i = acc_p[...] * inv_t
        f_i = acc_c[...] * (inv_t / _TOP_K)
        lb = jnp.sum(f_i * P_i, axis=-1, keepdims=True) * _NUM_EXPERTS
        prior = _DPSL_PRIOR
        dpsl = jnp.sum(prior * (jnp.log(prior) - jnp.log(P_i)),
                       axis=-1, keepdims=True)
        aux_ref[...] = _AUX_LOSS_WEIGHT * (lb + dpsl)


def kernel(x, W):
    b, s, d = x.shape
    total = b * s
    num_blocks = total // _BT
    xf = x.reshape(total, d)
    body = functools.partial(_router_body, num_blocks, total)
    probs, idx, topp, aux = pl.pallas_call(
        body,
        grid=(num_blocks,),
        in_specs=[
            pl.BlockSpec((_BT, d), lambda i: (i, 0)),
            pl.BlockSpec((d, _NUM_EXPERTS), lambda i: (0, 0)),
        ],
        out_specs=[
            pl.BlockSpec((_BT, _NUM_EXPERTS), lambda i: (i, 0)),
            pl.BlockSpec((_BT, _TOP_K), lambda i: (i, 0)),
            pl.BlockSpec((_BT, _TOP_K), lambda i: (i, 0)),
            pl.BlockSpec((1, 1), lambda i: (0, 0)),
        ],
        out_shape=[
            jax.ShapeDtypeStruct((total, _NUM_EXPERTS), jnp.float32),
            jax.ShapeDtypeStruct((total, _TOP_K), jnp.int32),
            jax.ShapeDtypeStruct((total, _TOP_K), jnp.float32),
            jax.ShapeDtypeStruct((1, 1), jnp.float32),
        ],
        scratch_shapes=[
            pltpu.VMEM((1, _NUM_EXPERTS), jnp.float32),
            pltpu.VMEM((1, _NUM_EXPERTS), jnp.float32),
        ],
        compiler_params=pltpu.CompilerParams(
            dimension_semantics=("arbitrary",),
        ),
    )(xf, W.T)
    return (probs.reshape(b, s, _NUM_EXPERTS),
            idx.reshape(b, s, _TOP_K),
            topp.reshape(b, s, _TOP_K),
            aux[0, 0])
